# Initial kernel scaffold; baseline (speedup 1.0000x reference)
import functools

import jax
import jax.numpy as jnp
from jax import lax
from jax.experimental import pallas as pl
from jax.experimental.pallas import tpu as pltpu
from jax.experimental.pallas import tpu_sc as plsc

N = 10000
E = 320000
D = 128

NC = 2
NS = 16
NW = NC * NS
EPW = E // NW
KB = 80
NBLK = EPW // KB
RPT = N // NS
ZROWS = 125
DEGW = 16


def _zero_vmem(buf, rows, cols):
    zeros16 = jnp.zeros((16,), jnp.float32)
    cpr = cols // 16

    def body(t, _):
        r = t // cpr
        c = (t % cpr) * 16
        buf[r, pl.ds(c, 16)] = zeros16
        return 0

    lax.fori_loop(0, rows * cpr, body, 0)


def _agg_body(h_hbm, src_hbm, dst_hbm, out_hbm, idx_s, idx_d, rows, zbuf,
              agg_sh, sem):
    c = lax.axis_index("c")
    s = lax.axis_index("s")
    w = c * NS + s
    ebase = w * EPW

    _zero_vmem(zbuf, ZROWS, D)
    for k in range(RPT // ZROWS):
        pltpu.sync_copy(zbuf, agg_sh.at[pl.ds(s * RPT + k * ZROWS, ZROWS)])
    plsc.subcore_barrier()

    def body(i, _):
        off = ebase + i * KB
        pltpu.sync_copy(src_hbm.at[pl.ds(off, KB)], idx_s)
        pltpu.sync_copy(dst_hbm.at[pl.ds(off, KB)], idx_d)
        pltpu.async_copy(h_hbm.at[idx_s], rows, sem).wait()
        pltpu.sync_copy(rows, agg_sh.at[idx_d], add=True)
        return 0

    lax.fori_loop(0, NBLK, body, 0)
    plsc.subcore_barrier()

    pltpu.sync_copy(agg_sh.at[pl.ds(s * RPT, RPT)],
                    out_hbm.at[c, pl.ds(s * RPT, RPT)])


_sc_agg = pl.kernel(
    _agg_body,
    out_type=jax.ShapeDtypeStruct((NC, N, D), jnp.float32),
    mesh=plsc.VectorSubcoreMesh(core_axis_name="c", subcore_axis_name="s"),
    scratch_types=[
        pltpu.VMEM((KB,), jnp.int32),
        pltpu.VMEM((KB,), jnp.int32),
        pltpu.VMEM((KB, D), jnp.float32),
        pltpu.VMEM((ZROWS, D), jnp.float32),
        pltpu.VMEM_SHARED((N, D), jnp.float32),
        pltpu.SemaphoreType.DMA,
    ],
)


def _deg_body(dst_hbm, out_hbm, idx_d, ones, zbuf, deg_sh, sem):
    c = lax.axis_index("c")
    s = lax.axis_index("s")
    w = c * NS + s
    ebase = w * EPW

    _zero_vmem(zbuf, ZROWS, DEGW)
    for k in range(RPT // ZROWS):
        pltpu.sync_copy(zbuf, deg_sh.at[pl.ds(s * RPT + k * ZROWS, ZROWS)])
    ones16 = jnp.ones((16,), jnp.float32)

    def fill(t, _):
        ones[t, pl.ds(0, 16)] = ones16
        return 0

    lax.fori_loop(0, KB, fill, 0)
    plsc.subcore_barrier()

    def body(i, _):
        off = ebase + i * KB
        pltpu.sync_copy(dst_hbm.at[pl.ds(off, KB)], idx_d)
        pltpu.sync_copy(ones, deg_sh.at[idx_d], add=True)
        return 0

    lax.fori_loop(0, NBLK, body, 0)
    plsc.subcore_barrier()
    pltpu.sync_copy(deg_sh.at[pl.ds(s * RPT, RPT)],
                    out_hbm.at[c, pl.ds(s * RPT, RPT)])


_sc_deg = pl.kernel(
    _deg_body,
    out_type=jax.ShapeDtypeStruct((NC, N, DEGW), jnp.float32),
    mesh=plsc.VectorSubcoreMesh(core_axis_name="c", subcore_axis_name="s"),
    scratch_types=[
        pltpu.VMEM((KB,), jnp.int32),
        pltpu.VMEM((KB, DEGW), jnp.float32),
        pltpu.VMEM((ZROWS, DEGW), jnp.float32),
        pltpu.VMEM_SHARED((N, DEGW), jnp.float32),
        pltpu.SemaphoreType.DMA,
    ],
)


NB = 1000


def _dense_block(p_ref, degp_ref, h_ref, wl_ref, wr_ref, b_ref, o_ref, *,
                 act):
    deg = degp_ref[0, :, 0] + degp_ref[1, :, 0]
    deginv = 1.0 / jnp.maximum(deg, 1.0)
    mean = (p_ref[0] + p_ref[1]) * deginv[:, None]
    y = (jnp.dot(mean, wl_ref[...], preferred_element_type=jnp.float32)
         + jnp.dot(h_ref[...], wr_ref[...], preferred_element_type=jnp.float32)
         + b_ref[...])
    if act == "relu":
        o_ref[...] = jnp.maximum(y, 0.0)
    else:
        m = jnp.max(y, axis=-1, keepdims=True)
        z = y - m
        lse = jnp.log(jnp.sum(jnp.exp(z), axis=-1, keepdims=True))
        o_ref[...] = z - lse


def _dense(p, degp, h, wl, wr, b, act):
    fn = pl.pallas_call(
        functools.partial(_dense_block, act=act),
        grid=(N // NB,),
        in_specs=[
            pl.BlockSpec((NC, NB, D), lambda i: (0, i, 0)),
            pl.BlockSpec((NC, NB, DEGW), lambda i: (0, i, 0)),
            pl.BlockSpec((NB, D), lambda i: (i, 0)),
            pl.BlockSpec((D, D), lambda i: (0, 0)),
            pl.BlockSpec((D, D), lambda i: (0, 0)),
            pl.BlockSpec((1, D), lambda i: (0, 0)),
        ],
        out_specs=pl.BlockSpec((NB, D), lambda i: (i, 0)),
        out_shape=jax.ShapeDtypeStruct((N, D), jnp.float32),
    )
    return fn(p, degp, h, wl, wr, b.reshape(1, D))


def kernel(x, adj_t, W1l, b1, W1r, W2l, b2, W2r, W3l, b3, W3r):
    src = adj_t[0]
    dst = adj_t[1]
    degp = _sc_deg(dst)
    p = _sc_agg(x, src, dst)
    h1 = _dense(p, degp, x, W1l, W1r, b1, "relu")
    p = _sc_agg(h1, src, dst)
    h2 = _dense(p, degp, h1, W2l, W2r, b2, "relu")
    p = _sc_agg(h2, src, dst)
    return _dense(p, degp, h2, W3l, W3r, b3, "logsoftmax")


# speedup vs baseline: 4.7758x; 4.7758x over previous
import functools

import jax
import jax.numpy as jnp
from jax import lax
from jax.experimental import pallas as pl
from jax.experimental.pallas import tpu as pltpu
from jax.experimental.pallas import tpu_sc as plsc

N = 10000
E = 320000
D = 128

NC = 2
NS = 16
NW = NC * NS
EPW = E // NW
KB = 80
NBLK = EPW // KB
NPAD = 10240
RPT = NPAD // NS
ZROWS = 128
DEGW = 16


def _zero_vmem(buf, rows, cols):
    zeros16 = jnp.zeros((16,), jnp.float32)
    cpr = cols // 16

    def body(t, _):
        r = t // cpr
        c = (t % cpr) * 16
        buf[r, pl.ds(c, 16)] = zeros16
        return 0

    lax.fori_loop(0, rows * cpr, body, 0)


def _agg_body(h_hbm, src_hbm, dst_hbm, out_hbm, idx_s, idx_d, rows, zbuf,
              agg_sh, sem):
    c = lax.axis_index("c")
    s = lax.axis_index("s")
    w = c * NS + s
    ebase = w * EPW

    _zero_vmem(zbuf, ZROWS, D)
    for k in range(RPT // ZROWS):
        pltpu.sync_copy(zbuf, agg_sh.at[pl.ds(s * RPT + k * ZROWS, ZROWS)])
    plsc.subcore_barrier()

    def body(i, _):
        off = ebase + i * KB
        pltpu.sync_copy(src_hbm.at[pl.ds(off, KB)], idx_s)
        pltpu.sync_copy(dst_hbm.at[pl.ds(off, KB)], idx_d)
        pltpu.async_copy(h_hbm.at[idx_s], rows, sem).wait()
        pltpu.sync_copy(rows, agg_sh.at[idx_d], add=True)
        return 0

    lax.fori_loop(0, NBLK, body, 0)
    plsc.subcore_barrier()

    pltpu.sync_copy(agg_sh.at[pl.ds(s * RPT, RPT)],
                    out_hbm.at[c, pl.ds(s * RPT, RPT)])


_sc_agg = pl.kernel(
    _agg_body,
    out_type=jax.ShapeDtypeStruct((NC, NPAD, D), jnp.float32),
    mesh=plsc.VectorSubcoreMesh(core_axis_name="c", subcore_axis_name="s"),
    scratch_types=[
        pltpu.VMEM((KB,), jnp.int32),
        pltpu.VMEM((KB,), jnp.int32),
        pltpu.VMEM((KB, D), jnp.float32),
        pltpu.VMEM((ZROWS, D), jnp.float32),
        pltpu.VMEM_SHARED((NPAD, D), jnp.float32),
        pltpu.SemaphoreType.DMA,
    ],
)


def _deg_body(dst_hbm, out_hbm, idx_d, ones, zbuf, deg_sh, sem):
    c = lax.axis_index("c")
    s = lax.axis_index("s")
    w = c * NS + s
    ebase = w * EPW

    _zero_vmem(zbuf, ZROWS, D)
    for k in range(RPT // ZROWS):
        pltpu.sync_copy(zbuf, deg_sh.at[pl.ds(s * RPT + k * ZROWS, ZROWS)])
    ones16 = jnp.ones((16,), jnp.float32)

    def fill(t, _):
        r = t // 8
        col = (t % 8) * 16
        ones[r, pl.ds(col, 16)] = ones16
        return 0

    lax.fori_loop(0, KB * 8, fill, 0)
    plsc.subcore_barrier()

    def body(i, _):
        off = ebase + i * KB
        pltpu.sync_copy(dst_hbm.at[pl.ds(off, KB)], idx_d)
        pltpu.sync_copy(ones, deg_sh.at[idx_d], add=True)
        return 0

    lax.fori_loop(0, NBLK, body, 0)
    plsc.subcore_barrier()
    pltpu.sync_copy(deg_sh.at[pl.ds(s * RPT, RPT)],
                    out_hbm.at[c, pl.ds(s * RPT, RPT)])


_sc_deg = pl.kernel(
    _deg_body,
    out_type=jax.ShapeDtypeStruct((NC, NPAD, D), jnp.float32),
    mesh=plsc.VectorSubcoreMesh(core_axis_name="c", subcore_axis_name="s"),
    scratch_types=[
        pltpu.VMEM((KB,), jnp.int32),
        pltpu.VMEM((KB, D), jnp.float32),
        pltpu.VMEM((ZROWS, D), jnp.float32),
        pltpu.VMEM_SHARED((NPAD, D), jnp.float32),
        pltpu.SemaphoreType.DMA,
    ],
)


NB = 1000


def _dense_block(p_ref, degp_ref, h_ref, wl_ref, wr_ref, b_ref, o_ref, *,
                 act):
    deg = degp_ref[0, :, 0] + degp_ref[1, :, 0]
    deginv = 1.0 / jnp.maximum(deg, 1.0)
    mean = (p_ref[0] + p_ref[1]) * deginv[:, None]
    y = (jnp.dot(mean, wl_ref[...], preferred_element_type=jnp.float32)
         + jnp.dot(h_ref[...], wr_ref[...], preferred_element_type=jnp.float32)
         + b_ref[...])
    if act == "relu":
        o_ref[...] = jnp.maximum(y, 0.0)
    else:
        m = jnp.max(y, axis=-1, keepdims=True)
        z = y - m
        lse = jnp.log(jnp.sum(jnp.exp(z), axis=-1, keepdims=True))
        o_ref[...] = z - lse


def _dense(p, degp, h, wl, wr, b, act):
    fn = pl.pallas_call(
        functools.partial(_dense_block, act=act),
        grid=(N // NB,),
        in_specs=[
            pl.BlockSpec((NC, NB, D), lambda i: (0, i, 0)),
            pl.BlockSpec((NC, NB, DEGW), lambda i: (0, i, 0)),
            pl.BlockSpec((NB, D), lambda i: (i, 0)),
            pl.BlockSpec((D, D), lambda i: (0, 0)),
            pl.BlockSpec((D, D), lambda i: (0, 0)),
            pl.BlockSpec((1, D), lambda i: (0, 0)),
        ],
        out_specs=pl.BlockSpec((NB, D), lambda i: (i, 0)),
        out_shape=jax.ShapeDtypeStruct((N, D), jnp.float32),
    )
    return fn(p, degp, h, wl, wr, b.reshape(1, D))


def kernel(x, adj_t, W1l, b1, W1r, W2l, b2, W2r, W3l, b3, W3r):
    src = adj_t[0]
    dst = adj_t[1]
    degp = _sc_deg(dst)[:, :, :DEGW]
    p = _sc_agg(x, src, dst)
    h1 = _dense(p, degp, x, W1l, W1r, b1, "relu")
    p = _sc_agg(h1, src, dst)
    h2 = _dense(p, degp, h1, W2l, W2r, b2, "relu")
    p = _sc_agg(h2, src, dst)
    return _dense(p, degp, h2, W3l, W3r, b3, "logsoftmax")


# speedup vs baseline: 8.8578x; 1.8547x over previous
import functools

import jax
import jax.numpy as jnp
from jax import lax
from jax.experimental import pallas as pl
from jax.experimental.pallas import tpu as pltpu
from jax.experimental.pallas import tpu_sc as plsc

N = 10000
E = 320000
D = 128

NC = 2
NS = 16
NW = NC * NS
EPW = E // NW
KB = 80
NBLK = EPW // KB
NPAD = 10240
RPT = NPAD // NS
DEGW = 16


def _zero_vmem(buf, rows, cols):
    zeros16 = jnp.zeros((16,), jnp.float32)
    cpr = cols // 16

    def body(t, _):
        r = t // cpr
        c = (t % cpr) * 16
        buf[r, pl.ds(c, 16)] = zeros16
        return 0

    lax.fori_loop(0, rows * cpr, body, 0)


def _agg_body(h_hbm, src_hbm, dst_hbm, out_hbm, sidx, didx, rows_a, rows_b,
              agg_sh, gsa, gsb, ssa, ssb):
    c = lax.axis_index("c")
    s = lax.axis_index("s")
    w = c * NS + s

    pltpu.async_copy(src_hbm.at[w], sidx, gsa)
    pltpu.async_copy(dst_hbm.at[w], didx, gsb)

    _zero_vmem(rows_a, KB, D)
    for k in range(RPT // KB):
        pltpu.sync_copy(rows_a, agg_sh.at[pl.ds(s * RPT + k * KB, KB)])
    pltpu.make_async_copy(src_hbm.at[w], sidx, gsa).wait()
    pltpu.make_async_copy(dst_hbm.at[w], didx, gsb).wait()
    plsc.subcore_barrier()

    pltpu.async_copy(h_hbm.at[sidx.at[pl.ds(0, KB)]], rows_a, gsa)

    def body(j, _):
        a = 2 * j
        b = a + 1
        pltpu.make_async_copy(h_hbm.at[sidx.at[pl.ds(a * KB, KB)]], rows_a,
                              gsa).wait()
        pltpu.async_copy(rows_a, agg_sh.at[didx.at[a]], ssa, add=True)
        pltpu.async_copy(h_hbm.at[sidx.at[pl.ds(b * KB, KB)]], rows_b, gsb)
        pltpu.make_async_copy(h_hbm.at[sidx.at[pl.ds(b * KB, KB)]], rows_b,
                              gsb).wait()
        pltpu.make_async_copy(rows_a, agg_sh.at[didx.at[a]], ssa).wait()
        pltpu.async_copy(rows_b, agg_sh.at[didx.at[b]], ssb, add=True)
        pltpu.async_copy(h_hbm.at[sidx.at[pl.ds((a + 2) * KB, KB)]],
                         rows_a, gsa)
        pltpu.make_async_copy(rows_b, agg_sh.at[didx.at[b]], ssb).wait()
        return 0

    lax.fori_loop(0, NBLK // 2, body, 0)
    last = NBLK - 1
    pltpu.make_async_copy(h_hbm.at[sidx.at[pl.ds(last * KB, KB)]], rows_a,
                          gsa).wait()
    pltpu.sync_copy(rows_a, agg_sh.at[didx.at[last]], add=True)
    plsc.subcore_barrier()

    pltpu.sync_copy(agg_sh.at[pl.ds(s * RPT, RPT)],
                    out_hbm.at[c, pl.ds(s * RPT, RPT)])


_sc_agg = pl.kernel(
    _agg_body,
    out_type=jax.ShapeDtypeStruct((NC, NPAD, D), jnp.float32),
    mesh=plsc.VectorSubcoreMesh(core_axis_name="c", subcore_axis_name="s"),
    scratch_types=[
        pltpu.VMEM((EPW,), jnp.int32),
        pltpu.VMEM((NBLK, KB), jnp.int32),
        pltpu.VMEM((KB, D), jnp.float32),
        pltpu.VMEM((KB, D), jnp.float32),
        pltpu.VMEM_SHARED((NPAD, D), jnp.float32),
        pltpu.SemaphoreType.DMA,
        pltpu.SemaphoreType.DMA,
        pltpu.SemaphoreType.DMA,
        pltpu.SemaphoreType.DMA,
    ],
)


def _deg_body(dst_hbm, out_hbm, didx, ones, deg_sh, gsa, ssa, ssb):
    c = lax.axis_index("c")
    s = lax.axis_index("s")
    w = c * NS + s

    pltpu.async_copy(dst_hbm.at[w], didx, gsa)
    _zero_vmem(ones, KB, D)
    for k in range(RPT // KB):
        pltpu.sync_copy(ones, deg_sh.at[pl.ds(s * RPT + k * KB, KB)])
    ones16 = jnp.ones((16,), jnp.float32)

    def fill(t, _):
        r = t // 8
        col = (t % 8) * 16
        ones[r, pl.ds(col, 16)] = ones16
        return 0

    lax.fori_loop(0, KB * 8, fill, 0)
    pltpu.make_async_copy(dst_hbm.at[w], didx, gsa).wait()
    plsc.subcore_barrier()

    def body(j, _):
        a = 2 * j
        b = a + 1
        pltpu.async_copy(ones, deg_sh.at[didx.at[a]], ssa, add=True)
        pltpu.async_copy(ones, deg_sh.at[didx.at[b]], ssb, add=True)
        pltpu.make_async_copy(ones, deg_sh.at[didx.at[a]], ssa).wait()
        pltpu.make_async_copy(ones, deg_sh.at[didx.at[b]], ssb).wait()
        return 0

    lax.fori_loop(0, NBLK // 2, body, 0)
    pltpu.sync_copy(ones, deg_sh.at[didx.at[NBLK - 1]], add=True)
    plsc.subcore_barrier()
    pltpu.sync_copy(deg_sh.at[pl.ds(s * RPT, RPT)],
                    out_hbm.at[c, pl.ds(s * RPT, RPT)])


_sc_deg = pl.kernel(
    _deg_body,
    out_type=jax.ShapeDtypeStruct((NC, NPAD, D), jnp.float32),
    mesh=plsc.VectorSubcoreMesh(core_axis_name="c", subcore_axis_name="s"),
    scratch_types=[
        pltpu.VMEM((NBLK, KB), jnp.int32),
        pltpu.VMEM((KB, D), jnp.float32),
        pltpu.VMEM_SHARED((NPAD, D), jnp.float32),
        pltpu.SemaphoreType.DMA,
        pltpu.SemaphoreType.DMA,
        pltpu.SemaphoreType.DMA,
    ],
)


NB = 1000


def _dense_block(p_ref, degp_ref, h_ref, wl_ref, wr_ref, b_ref, o_ref, *,
                 act):
    deg = degp_ref[0, :, 0] + degp_ref[1, :, 0]
    deginv = 1.0 / jnp.maximum(deg, 1.0)
    mean = (p_ref[0] + p_ref[1]) * deginv[:, None]
    y = (jnp.dot(mean, wl_ref[...], preferred_element_type=jnp.float32)
         + jnp.dot(h_ref[...], wr_ref[...], preferred_element_type=jnp.float32)
         + b_ref[...])
    if act == "relu":
        o_ref[...] = jnp.maximum(y, 0.0)
    else:
        m = jnp.max(y, axis=-1, keepdims=True)
        z = y - m
        lse = jnp.log(jnp.sum(jnp.exp(z), axis=-1, keepdims=True))
        o_ref[...] = z - lse


def _dense(p, degp, h, wl, wr, b, act):
    fn = pl.pallas_call(
        functools.partial(_dense_block, act=act),
        grid=(N // NB,),
        in_specs=[
            pl.BlockSpec((NC, NB, D), lambda i: (0, i, 0)),
            pl.BlockSpec((NC, NB, DEGW), lambda i: (0, i, 0)),
            pl.BlockSpec((NB, D), lambda i: (i, 0)),
            pl.BlockSpec((D, D), lambda i: (0, 0)),
            pl.BlockSpec((D, D), lambda i: (0, 0)),
            pl.BlockSpec((1, D), lambda i: (0, 0)),
        ],
        out_specs=pl.BlockSpec((NB, D), lambda i: (i, 0)),
        out_shape=jax.ShapeDtypeStruct((N, D), jnp.float32),
    )
    return fn(p, degp, h, wl, wr, b.reshape(1, D))


def kernel(x, adj_t, W1l, b1, W1r, W2l, b2, W2r, W3l, b3, W3r):
    src = adj_t[0].reshape(NW, EPW)
    dst = adj_t[1].reshape(NW, NBLK, KB)
    degp = _sc_deg(dst)[:, :, :DEGW]
    p = _sc_agg(x, src, dst)
    h1 = _dense(p, degp, x, W1l, W1r, b1, "relu")
    p = _sc_agg(h1, src, dst)
    h2 = _dense(p, degp, h1, W2l, W2r, b2, "relu")
    p = _sc_agg(h2, src, dst)
    return _dense(p, degp, h2, W3l, W3r, b3, "logsoftmax")


# speedup vs baseline: 8.9870x; 1.0146x over previous
import functools

import jax
import jax.numpy as jnp
from jax import lax
from jax.experimental import pallas as pl
from jax.experimental.pallas import tpu as pltpu
from jax.experimental.pallas import tpu_sc as plsc

N = 10000
E = 320000
D = 128

NC = 2
NS = 16
NW = NC * NS
EPW = E // NW
KB = 80
NBLK = EPW // KB
NPAD = 10240
RPT = NPAD // NS
DEGW = 16


def _zero_vmem(buf, rows, cols):
    zeros16 = jnp.zeros((16,), jnp.float32)
    cpr = cols // 16

    def body(t, _):
        r = t // cpr
        c = (t % cpr) * 16
        buf[r, pl.ds(c, 16)] = zeros16
        return 0

    lax.fori_loop(0, rows * cpr, body, 0)


def _agg_body(h_hbm, src_hbm, dst_hbm, out_hbm, sidx, didx, rows_a, rows_b,
              agg_sh, gsa, gsb, ssa, ssb):
    c = lax.axis_index("c")
    s = lax.axis_index("s")
    w = c * NS + s

    pltpu.async_copy(src_hbm.at[w], sidx, gsa)
    pltpu.async_copy(dst_hbm.at[w], didx, gsb)

    _zero_vmem(rows_a, KB, D)
    for k in range(RPT // KB):
        pltpu.sync_copy(rows_a, agg_sh.at[pl.ds(s * RPT + k * KB, KB)])
    pltpu.make_async_copy(src_hbm.at[w], sidx, gsa).wait()
    pltpu.make_async_copy(dst_hbm.at[w], didx, gsb).wait()
    plsc.subcore_barrier()

    pltpu.async_copy(h_hbm.at[sidx.at[pl.ds(0, KB)]], rows_a, gsa)

    def body(j, _):
        a = 2 * j
        b = a + 1
        pltpu.make_async_copy(h_hbm.at[sidx.at[pl.ds(a * KB, KB)]], rows_a,
                              gsa).wait()
        pltpu.async_copy(rows_a, agg_sh.at[didx.at[a]], ssa, add=True)
        pltpu.async_copy(h_hbm.at[sidx.at[pl.ds(b * KB, KB)]], rows_b, gsb)
        pltpu.make_async_copy(h_hbm.at[sidx.at[pl.ds(b * KB, KB)]], rows_b,
                              gsb).wait()
        pltpu.async_copy(rows_b, agg_sh.at[didx.at[b]], ssb, add=True)
        pltpu.make_async_copy(rows_a, agg_sh.at[didx.at[a]], ssa).wait()
        pltpu.async_copy(h_hbm.at[sidx.at[pl.ds((a + 2) * KB, KB)]],
                         rows_a, gsa)
        pltpu.make_async_copy(rows_b, agg_sh.at[didx.at[b]], ssb).wait()
        return 0

    lax.fori_loop(0, NBLK // 2, body, 0)
    last = NBLK - 1
    pltpu.make_async_copy(h_hbm.at[sidx.at[pl.ds(last * KB, KB)]], rows_a,
                          gsa).wait()
    pltpu.sync_copy(rows_a, agg_sh.at[didx.at[last]], add=True)
    plsc.subcore_barrier()

    pltpu.sync_copy(agg_sh.at[pl.ds(s * RPT, RPT)],
                    out_hbm.at[c, pl.ds(s * RPT, RPT)])


_sc_agg = pl.kernel(
    _agg_body,
    out_type=jax.ShapeDtypeStruct((NC, NPAD, D), jnp.float32),
    mesh=plsc.VectorSubcoreMesh(core_axis_name="c", subcore_axis_name="s"),
    scratch_types=[
        pltpu.VMEM((EPW,), jnp.int32),
        pltpu.VMEM((NBLK, KB), jnp.int32),
        pltpu.VMEM((KB, D), jnp.float32),
        pltpu.VMEM((KB, D), jnp.float32),
        pltpu.VMEM_SHARED((NPAD, D), jnp.float32),
        pltpu.SemaphoreType.DMA,
        pltpu.SemaphoreType.DMA,
        pltpu.SemaphoreType.DMA,
        pltpu.SemaphoreType.DMA,
    ],
)


def _deg_body(dst_hbm, out_hbm, didx, ones, deg_sh, gsa, ssa, ssb):
    c = lax.axis_index("c")
    s = lax.axis_index("s")
    w = c * NS + s

    pltpu.async_copy(dst_hbm.at[w], didx, gsa)
    _zero_vmem(ones, KB, D)
    for k in range(RPT // KB):
        pltpu.sync_copy(ones, deg_sh.at[pl.ds(s * RPT + k * KB, KB)])
    ones16 = jnp.ones((16,), jnp.float32)

    def fill(t, _):
        r = t // 8
        col = (t % 8) * 16
        ones[r, pl.ds(col, 16)] = ones16
        return 0

    lax.fori_loop(0, KB * 8, fill, 0)
    pltpu.make_async_copy(dst_hbm.at[w], didx, gsa).wait()
    plsc.subcore_barrier()

    def body(j, _):
        a = 2 * j
        b = a + 1
        pltpu.async_copy(ones, deg_sh.at[didx.at[a]], ssa, add=True)
        pltpu.async_copy(ones, deg_sh.at[didx.at[b]], ssb, add=True)
        pltpu.make_async_copy(ones, deg_sh.at[didx.at[a]], ssa).wait()
        pltpu.make_async_copy(ones, deg_sh.at[didx.at[b]], ssb).wait()
        return 0

    lax.fori_loop(0, NBLK // 2, body, 0)
    pltpu.sync_copy(ones, deg_sh.at[didx.at[NBLK - 1]], add=True)
    plsc.subcore_barrier()
    pltpu.sync_copy(deg_sh.at[pl.ds(s * RPT, RPT)],
                    out_hbm.at[c, pl.ds(s * RPT, RPT)])


_sc_deg = pl.kernel(
    _deg_body,
    out_type=jax.ShapeDtypeStruct((NC, NPAD, D), jnp.float32),
    mesh=plsc.VectorSubcoreMesh(core_axis_name="c", subcore_axis_name="s"),
    scratch_types=[
        pltpu.VMEM((NBLK, KB), jnp.int32),
        pltpu.VMEM((KB, D), jnp.float32),
        pltpu.VMEM_SHARED((NPAD, D), jnp.float32),
        pltpu.SemaphoreType.DMA,
        pltpu.SemaphoreType.DMA,
        pltpu.SemaphoreType.DMA,
    ],
)


NB = 2000


def _dense_block(p_ref, degp_ref, h_ref, wl_ref, wr_ref, b_ref, o_ref, *,
                 act):
    deg = degp_ref[0, :, 0] + degp_ref[1, :, 0]
    deginv = 1.0 / jnp.maximum(deg, 1.0)
    mean = (p_ref[0] + p_ref[1]) * deginv[:, None]
    y = (jnp.dot(mean, wl_ref[...], preferred_element_type=jnp.float32)
         + jnp.dot(h_ref[...], wr_ref[...], preferred_element_type=jnp.float32)
         + b_ref[...])
    if act == "relu":
        o_ref[...] = jnp.maximum(y, 0.0)
    else:
        m = jnp.max(y, axis=-1, keepdims=True)
        z = y - m
        lse = jnp.log(jnp.sum(jnp.exp(z), axis=-1, keepdims=True))
        o_ref[...] = z - lse


def _dense(p, degp, h, wl, wr, b, act):
    fn = pl.pallas_call(
        functools.partial(_dense_block, act=act),
        grid=(N // NB,),
        in_specs=[
            pl.BlockSpec((NC, NB, D), lambda i: (0, i, 0)),
            pl.BlockSpec((NC, NB, DEGW), lambda i: (0, i, 0)),
            pl.BlockSpec((NB, D), lambda i: (i, 0)),
            pl.BlockSpec((D, D), lambda i: (0, 0)),
            pl.BlockSpec((D, D), lambda i: (0, 0)),
            pl.BlockSpec((1, D), lambda i: (0, 0)),
        ],
        out_specs=pl.BlockSpec((NB, D), lambda i: (i, 0)),
        out_shape=jax.ShapeDtypeStruct((N, D), jnp.float32),
    )
    return fn(p, degp, h, wl, wr, b.reshape(1, D))


def kernel(x, adj_t, W1l, b1, W1r, W2l, b2, W2r, W3l, b3, W3r):
    src = adj_t[0].reshape(NW, EPW)
    dst = adj_t[1].reshape(NW, NBLK, KB)
    degp = _sc_deg(dst)[:, :, :DEGW]
    p = _sc_agg(x, src, dst)
    h1 = _dense(p, degp, x, W1l, W1r, b1, "relu")
    p = _sc_agg(h1, src, dst)
    h2 = _dense(p, degp, h1, W2l, W2r, b2, "relu")
    p = _sc_agg(h2, src, dst)
    return _dense(p, degp, h2, W3l, W3r, b3, "logsoftmax")
